# baseline (speedup 1.0000x reference)
import functools

import jax
import jax.numpy as jnp
from jax import lax
from jax.experimental import pallas as pl
from jax.experimental.pallas import tpu as pltpu
from jax.experimental.pallas import tpu_sc as plsc

N_NODES = 10000
N_EDGES = 320000
E_ROWS = 2500
NP = 10240
NPT = 640
ROWS = 160
TAIL_ROWS = 100
LANES = 16


def _rsqrt16(x):
    i = lax.bitcast_convert_type(x, jnp.int32)
    i = 0x5F3759DF - lax.shift_right_arithmetic(i, 1)
    y = lax.bitcast_convert_type(i, jnp.float32)
    for _ in range(4):
        y = y * (1.5 - 0.5 * x * y * y)
    return y


def _tc_matvec(x_ref, w1_ref, w2_ref, wfc_ref, b1_ref, b2_ref, bfc_ref,
               z_ref, c_ref):
    wv = jnp.dot(w2_ref[...], wfc_ref[...], preferred_element_type=jnp.float32)
    w = jnp.dot(w1_ref[...], wv, preferred_element_type=jnp.float32)
    z_ref[...] = jnp.dot(x_ref[...], w, preferred_element_type=jnp.float32)
    c1 = jnp.sum(b1_ref[...] * wv[:, 0][None, :])
    c2 = jnp.sum(b2_ref[...] * wfc_ref[...][:, 0][None, :]) + bfc_ref[0, 0]
    lane = lax.broadcasted_iota(jnp.int32, (1, 128), 1)
    c_ref[...] = jnp.where(lane < LANES, c1, c2)


def _sc_body(src2d, dst2d, ew2d, z_hbm, consts, out_hbm,
             esrc, edst, eww, vals, ufull, dis, sbuf, tbuf, zbuf, cvec,
             shacc, shpub, sem_s):
    s = lax.axis_index("s")
    c = lax.axis_index("c")
    own = pl.ds(s * NPT, NPT)
    row0 = s * ROWS
    nrows = jnp.where(s < 15, ROWS, TAIL_ROWS)
    zero16 = jnp.zeros((LANES,), jnp.float32)

    @pl.when(s < 15)
    def _():
        pltpu.sync_copy(src2d.at[pl.ds(row0, ROWS)], esrc)
        pltpu.sync_copy(dst2d.at[pl.ds(row0, ROWS)], edst)
        pltpu.sync_copy(ew2d.at[pl.ds(row0, ROWS)], eww)

    @pl.when(s == 15)
    def _():
        pltpu.sync_copy(src2d.at[pl.ds(15 * ROWS, TAIL_ROWS)],
                        esrc.at[pl.ds(0, TAIL_ROWS)])
        pltpu.sync_copy(dst2d.at[pl.ds(15 * ROWS, TAIL_ROWS)],
                        edst.at[pl.ds(0, TAIL_ROWS)])
        pltpu.sync_copy(ew2d.at[pl.ds(15 * ROWS, TAIL_ROWS)],
                        eww.at[pl.ds(0, TAIL_ROWS)])

    @pl.when(s < 15)
    def _():
        pltpu.sync_copy(z_hbm.at[own], sbuf)

    @pl.when(s == 15)
    def _():
        pltpu.sync_copy(z_hbm.at[pl.ds(15 * NPT, N_NODES - 15 * NPT)],
                        sbuf.at[pl.ds(0, N_NODES - 15 * NPT)])
        for i in range(25, 40):
            sbuf[pl.ds(i * LANES, LANES)] = zero16

    pltpu.sync_copy(consts, cvec)

    def _zero(i, carry):
        zbuf[pl.ds(i * LANES, LANES)] = zero16
        return carry
    lax.fori_loop(0, NPT // LANES, _zero, 0)

    pltpu.sync_copy(zbuf, shacc.at[own])
    plsc.subcore_barrier()

    def _drain_pass():
        @pl.when(s < 15)
        def _():
            pltpu.make_async_copy(ew2d.at[pl.ds(0, ROWS)], vals, sem_s).wait()

        @pl.when(s == 15)
        def _():
            pltpu.make_async_copy(ew2d.at[pl.ds(0, TAIL_ROWS)],
                                  vals.at[pl.ds(0, TAIL_ROWS)], sem_s).wait()
        plsc.subcore_barrier()

    def _deg_row(j, carry):
        pltpu.async_copy(eww.at[j], shacc.at[edst.at[j]], sem_s, add=True)
        return carry
    lax.fori_loop(0, nrows, _deg_row, 0)
    _drain_pass()

    pltpu.sync_copy(shacc.at[own], tbuf)

    def _dis(i, carry):
        sl = pl.ds(i * LANES, LANES)
        d = _rsqrt16(tbuf[sl] + 1.0)
        dis[sl] = d
        sbuf[sl] = d * sbuf[sl]
        return carry
    lax.fori_loop(0, NPT // LANES, _dis, 0)
    pltpu.sync_copy(sbuf, shpub.at[own])
    pltpu.sync_copy(zbuf, shacc.at[own])
    plsc.subcore_barrier()
    pltpu.sync_copy(shpub, ufull)

    def _edge_pass(j, carry):
        for k in range(8):
            sl = pl.ds(k * LANES, LANES)
            vals[j, sl] = eww[j, sl] * plsc.load_gather(ufull, [esrc[j, sl]])
        pltpu.async_copy(vals.at[j], shacc.at[edst.at[j]], sem_s, add=True)
        return carry
    lax.fori_loop(0, nrows, _edge_pass, 0)
    _drain_pass()

    pltpu.sync_copy(shacc.at[own], tbuf)
    c1v = cvec[0, pl.ds(0, LANES)]

    def _fin1(i, carry):
        sl = pl.ds(i * LANES, LANES)
        d = dis[sl]
        sbuf[sl] = d * d * (tbuf[sl] + sbuf[sl]) + c1v * d
        return carry
    lax.fori_loop(0, NPT // LANES, _fin1, 0)
    pltpu.sync_copy(sbuf, shpub.at[own])
    pltpu.sync_copy(zbuf, shacc.at[own])
    plsc.subcore_barrier()
    pltpu.sync_copy(shpub, ufull)

    lax.fori_loop(0, nrows, _edge_pass, 0)
    _drain_pass()

    pltpu.sync_copy(shacc.at[own], tbuf)
    c2v = cvec[0, pl.ds(LANES, LANES)]

    def _fin2(i, carry):
        sl = pl.ds(i * LANES, LANES)
        t = dis[sl] * (tbuf[sl] + sbuf[sl]) + c2v
        sbuf[sl] = 10.0 / (1.0 + jnp.exp(-t))
        return carry
    lax.fori_loop(0, NPT // LANES, _fin2, 0)

    @pl.when(c == 0)
    def _():
        pltpu.sync_copy(sbuf, out_hbm.at[own])


_sc_agg = functools.partial(
    pl.kernel,
    out_type=jax.ShapeDtypeStruct((NP,), jnp.float32),
    mesh=plsc.VectorSubcoreMesh(core_axis_name="c", subcore_axis_name="s"),
    compiler_params=pltpu.CompilerParams(needs_layout_passes=False,
                                         use_tc_tiling_on_sc=False),
    scratch_types=[
        pltpu.VMEM((ROWS, 128), jnp.int32),
        pltpu.VMEM((ROWS, 128), jnp.int32),
        pltpu.VMEM((ROWS, 128), jnp.float32),
        pltpu.VMEM((ROWS, 128), jnp.float32),
        pltpu.VMEM((NP,), jnp.float32),
        pltpu.VMEM((NPT,), jnp.float32),
        pltpu.VMEM((NPT,), jnp.float32),
        pltpu.VMEM((NPT,), jnp.float32),
        pltpu.VMEM((NPT,), jnp.float32),
        pltpu.VMEM((1, 128), jnp.float32),
        pltpu.VMEM_SHARED((NP,), jnp.float32),
        pltpu.VMEM_SHARED((NP,), jnp.float32),
        pltpu.SemaphoreType.DMA,
    ],
)(_sc_body)


@jax.jit
def kernel(x, edge_index, edge_weight, W1, b1, W2, b2, Wfc, bfc):
    src2d = edge_index[0].astype(jnp.int32).reshape(E_ROWS, 128)
    dst2d = edge_index[1].astype(jnp.int32).reshape(E_ROWS, 128)
    ew2d = edge_weight.astype(jnp.float32).reshape(E_ROWS, 128)

    grid = 10
    rows_blk = N_NODES // grid
    z2d, cvec = pl.pallas_call(
        _tc_matvec,
        grid=(grid,),
        in_specs=[
            pl.BlockSpec((rows_blk, 128), lambda i: (i, 0)),
            pl.BlockSpec((128, 128), lambda i: (0, 0)),
            pl.BlockSpec((128, 128), lambda i: (0, 0)),
            pl.BlockSpec((128, 1), lambda i: (0, 0)),
            pl.BlockSpec((1, 128), lambda i: (0, 0)),
            pl.BlockSpec((1, 128), lambda i: (0, 0)),
            pl.BlockSpec((1, 1), lambda i: (0, 0)),
        ],
        out_specs=[
            pl.BlockSpec((rows_blk, 1), lambda i: (i, 0)),
            pl.BlockSpec((1, 128), lambda i: (0, 0)),
        ],
        out_shape=[
            jax.ShapeDtypeStruct((N_NODES, 1), jnp.float32),
            jax.ShapeDtypeStruct((1, 128), jnp.float32),
        ],
    )(x, W1, W2, Wfc, b1[None, :], b2[None, :], bfc[None, :])

    out_pad = _sc_agg(src2d, dst2d, ew2d, z2d[:, 0], cvec)
    return out_pad[:N_NODES, None]
